# baseline (speedup 1.0000x reference)
import functools

import jax
import jax.numpy as jnp
from jax import lax
from jax.experimental import pallas as pl
from jax.experimental.pallas import tpu as pltpu
from jax.experimental.pallas import tpu_sc as plsc

N_NODES = 10000
D_FEAT = 128
D_HID = 16
N_EDGES = 320000

NC = 2
NS = 16
NW = NC * NS
CHUNK = 128
NCH = 80
NCHT = NW * NCH
E_PAD = NCHT * CHUNK
NPA = N_NODES + 16
RPS = N_NODES // NS
G = 4
NGRP = NCH // G
HALF = 5000
PK = N_NODES // 8
PKH = HALF // 8


def _vector_mesh():
    return plsc.VectorSubcoreMesh(
        core_axis_name="c", subcore_axis_name="s", num_cores=NC, num_subcores=NS
    )


def _sc_degree(dstb, ones_hbm, zeros_hbm):
    @functools.partial(
        pl.kernel,
        out_type=jax.ShapeDtypeStruct((NC, N_NODES, D_HID), jnp.float32),
        mesh=_vector_mesh(),
        compiler_params=pltpu.CompilerParams(use_tc_tiling_on_sc=False),
        scratch_types=[
            pltpu.VMEM((NCH, CHUNK), jnp.int32),
            pltpu.VMEM((CHUNK, D_HID), jnp.float32),
            pltpu.VMEM_SHARED((NPA, D_HID), jnp.float32),
            pltpu.SemaphoreType.DMA,
        ],
    )
    def deg_kernel(dstb_hbm, ones_h, zeros_h, out_hbm, didx_v, ones_v, acc_s,
                   sem):
        c = lax.axis_index("c")
        s = lax.axis_index("s")
        wid = s * NC + c
        r0 = s * RPS
        st = [
            pltpu.async_copy(zeros_h.at[pl.ds(0, RPS)],
                             acc_s.at[pl.ds(r0, RPS)], sem),
            pltpu.async_copy(ones_h, ones_v, sem),
            pltpu.async_copy(dstb_hbm.at[pl.ds(wid * NCH, NCH)],
                             didx_v, sem),
        ]
        for cp in st:
            cp.wait()
        plsc.subcore_barrier()

        def fire8(g):
            for k in range(8):
                pltpu.async_copy(ones_v, acc_s.at[didx_v.at[g * 8 + k]], sem,
                                 add=True)

        def drain(n):
            for _ in range(n):
                pltpu.make_async_copy(out_hbm.at[0].at[pl.ds(0, CHUNK)],
                                      ones_v, sem).wait()

        fire8(0)
        fire8(1)

        @pl.loop(0, NCH // 8 - 2)
        def _(g):
            fire8(g + 2)
            drain(8)

        drain(16)
        plsc.subcore_barrier()
        pltpu.sync_copy(acc_s.at[pl.ds(r0, RPS)],
                        out_hbm.at[c].at[pl.ds(r0, RPS)])

    return deg_kernel(dstb, ones_hbm, zeros_hbm)


def _sc_aggregate(hs_lin, srcb, dstb, zeros_hbm):
    @functools.partial(
        pl.kernel,
        out_type=jax.ShapeDtypeStruct((NC, N_NODES, D_HID), jnp.float32),
        mesh=_vector_mesh(),
        compiler_params=pltpu.CompilerParams(use_tc_tiling_on_sc=False),
        scratch_types=[
            pltpu.VMEM((NCH, CHUNK), jnp.int32),
            pltpu.VMEM((NCH, CHUNK), jnp.int32),
            pltpu.VMEM((G, CHUNK, D_HID), jnp.float32),
            pltpu.VMEM((G, CHUNK, D_HID), jnp.float32),
            pltpu.VMEM_SHARED((NPA, D_HID), jnp.float32),
            pltpu.VMEM_SHARED((NPA, D_HID), jnp.float32),
            pltpu.SemaphoreType.DMA,
            pltpu.SemaphoreType.DMA,
            pltpu.SemaphoreType.DMA,
            pltpu.SemaphoreType.DMA,
        ],
    )
    def agg_kernel(hs_hbm, srcb_hbm, dstb_hbm, zeros_h, out_hbm,
                   sidx_v, didx_v, rows_a, rows_b, hs_s, acc_s,
                   semga, semgb, semsa, semsb):
        c = lax.axis_index("c")
        s = lax.axis_index("s")
        wid = s * NC + c
        r0 = s * RPS
        st = [
            pltpu.async_copy(hs_hbm.at[pl.ds(r0, RPS)],
                             hs_s.at[pl.ds(r0, RPS)], semga),
            pltpu.async_copy(zeros_h.at[pl.ds(0, RPS)],
                             acc_s.at[pl.ds(r0, RPS)], semgb),
            pltpu.async_copy(srcb_hbm.at[pl.ds(wid * NCH, NCH)], sidx_v,
                             semsa),
            pltpu.async_copy(dstb_hbm.at[pl.ds(wid * NCH, NCH)],
                             didx_v, semsb),
        ]
        for cp in st:
            cp.wait()
        plsc.subcore_barrier()

        def fire_g(g, rows, sem):
            for k in range(G):
                pltpu.async_copy(hs_s.at[sidx_v.at[g * G + k]], rows.at[k],
                                 sem)

        def fire_s(g, rows, sem):
            for k in range(G):
                pltpu.async_copy(rows.at[k], acc_s.at[didx_v.at[g * G + k]],
                                 sem, add=True)

        def drain(sem, n):
            for _ in range(n):
                pltpu.make_async_copy(hs_hbm.at[pl.ds(0, CHUNK)],
                                      rows_a.at[0], sem).wait()

        fire_g(0, rows_a, semga)
        fire_g(1, rows_b, semgb)
        drain(semga, G)
        fire_s(0, rows_a, semsa)

        @pl.loop(1, NGRP - 1, step=2)
        def _(g):
            drain(semsa, G)
            fire_g(g + 1, rows_a, semga)
            drain(semgb, G)
            fire_s(g, rows_b, semsb)
            drain(semsb, G)
            fire_g(g + 2, rows_b, semgb)
            drain(semga, G)
            fire_s(g + 1, rows_a, semsa)

        drain(semsa, G)
        drain(semgb, G)
        fire_s(NGRP - 1, rows_b, semsb)
        drain(semsb, G)

        plsc.subcore_barrier()
        pltpu.sync_copy(acc_s.at[pl.ds(r0, RPS)],
                        out_hbm.at[c].at[pl.ds(r0, RPS)])

    return agg_kernel(hs_lin, srcb, dstb, zeros_hbm)


def _pk_spec():
    return pl.BlockSpec((PK, D_FEAT), lambda i: (0, 0))


def _pair_spec():
    return pl.BlockSpec((NC, PK, D_FEAT), lambda i: (0, 0, 0))


def _full_spec(shape):
    return pl.BlockSpec(shape, lambda i: tuple(0 for _ in shape))


def _mm1_body(x_ref, g_ref, w_ref, h_ref):
    x3 = x_ref[...]
    w = w_ref[...]
    parts = [
        jnp.dot(x3[:, j, :], w, preferred_element_type=jnp.float32)
        for j in range(8)
    ]
    h_ref[...] = jnp.concatenate(parts, axis=1) * g_ref[...]


def _scale_body(degp_ref, h_ref, dinv_ref, hs_ref):
    deg = degp_ref[0] + degp_ref[1] + 1.0
    dinv = lax.rsqrt(deg)
    dinv_ref[...] = dinv
    hs_ref[...] = h_ref[...] * dinv


def _relu_body(aggp_ref, hs1_ref, dinv_ref, b1_ref, hs2_ref):
    dinv = dinv_ref[...]
    conv1 = dinv * (aggp_ref[0] + aggp_ref[1] + hs1_ref[...]) + b1_ref[...]
    hs2_ref[...] = jnp.maximum(conv1, 0.0) * dinv


def _mm2_body(aggp_ref, hs2_ref, dinv_ref, w_ref, b_ref, o_ref):
    agg2 = dinv_ref[...] * (aggp_ref[0] + aggp_ref[1] + hs2_ref[...])
    w = w_ref[...]
    b = b_ref[...]
    for j in range(8):
        aj = agg2[:, 16 * j:16 * j + 16]
        o_ref[:, j, :] = (
            jnp.dot(aj, w, preferred_element_type=jnp.float32) + b
        )


def kernel(x, graph_seq, edge_index, W1, b1, W2, b2):
    nb, nc_, nd = x.shape
    n = nb * nc_

    xt = x.transpose(1, 0, 2)
    gs_pk = jnp.repeat(graph_seq.transpose(1, 0), D_HID, axis=1)

    pad_e = E_PAD - N_EDGES
    ei = edge_index.astype(jnp.int32)
    ei = (ei % nc_) * nb + ei // nc_
    dstb = jnp.pad(ei[1], (0, pad_e), constant_values=N_NODES)
    dstb = dstb.reshape(NCHT, CHUNK)
    srcb = jnp.pad(ei[0], (0, pad_e)).reshape(NCHT, CHUNK)

    zeros = jnp.zeros((RPS, D_HID), jnp.float32)
    ones = jnp.ones((CHUNK, D_HID), jnp.float32)
    b1t = jnp.tile(b1, 8).reshape(1, D_FEAT)
    b2r = b2.reshape(1, D_FEAT)

    degp = _sc_degree(dstb, ones, zeros).reshape(NC, PK, D_FEAT)

    h1 = pl.pallas_call(
        _mm1_body,
        grid=(1,),
        in_specs=[
            pl.BlockSpec((PK, nb, nd), lambda i: (0, 0, 0)),
            _full_spec((PK, D_FEAT)),
            _full_spec((D_FEAT, D_HID)),
        ],
        out_specs=_pk_spec(),
        out_shape=jax.ShapeDtypeStruct((PK, D_FEAT), jnp.float32),
    )(xt, gs_pk, W1)

    dinv, hs1 = pl.pallas_call(
        _scale_body,
        grid=(1,),
        in_specs=[_pair_spec(), _pk_spec()],
        out_specs=(_pk_spec(), _pk_spec()),
        out_shape=(
            jax.ShapeDtypeStruct((PK, D_FEAT), jnp.float32),
            jax.ShapeDtypeStruct((PK, D_FEAT), jnp.float32),
        ),
    )(degp, h1)

    agg1 = _sc_aggregate(hs1.reshape(n, D_HID), srcb, dstb, zeros)
    agg1 = agg1.reshape(NC, PK, D_FEAT)

    hs2 = pl.pallas_call(
        _relu_body,
        grid=(1,),
        in_specs=[_pair_spec(), _pk_spec(), _pk_spec(),
                  _full_spec((1, D_FEAT))],
        out_specs=_pk_spec(),
        out_shape=jax.ShapeDtypeStruct((PK, D_FEAT), jnp.float32),
    )(agg1, hs1, dinv, b1t)

    agg2 = _sc_aggregate(hs2.reshape(n, D_HID), srcb, dstb, zeros)
    agg2 = agg2.reshape(NC, PK, D_FEAT)

    out_t = pl.pallas_call(
        _mm2_body,
        grid=(1,),
        in_specs=[_pair_spec(), _pk_spec(), _pk_spec(),
                  _full_spec((D_HID, D_FEAT)), _full_spec((1, D_FEAT))],
        out_specs=pl.BlockSpec((PK, nb, nd), lambda i: (0, 0, 0)),
        out_shape=jax.ShapeDtypeStruct((PK, nb, nd), jnp.float32),
    )(agg2, hs2, dinv, W2, b2r)

    return out_t.transpose(1, 0, 2)


# speedup vs baseline: 1.1018x; 1.1018x over previous
import functools

import jax
import jax.numpy as jnp
from jax import lax
from jax.experimental import pallas as pl
from jax.experimental.pallas import tpu as pltpu
from jax.experimental.pallas import tpu_sc as plsc

N_NODES = 10000
D_FEAT = 128
D_HID = 16
N_EDGES = 320000

NC = 2
NS = 16
NW = NC * NS
CHUNK = 128
NCH = 80
NCHT = NW * NCH
E_PAD = NCHT * CHUNK
NPA = N_NODES + 16
RPS = N_NODES // NS
G = 4
NGRP = NCH // G
HALF = 5000
PK = N_NODES // 8
PKH = HALF // 8


def _vector_mesh():
    return plsc.VectorSubcoreMesh(
        core_axis_name="c", subcore_axis_name="s", num_cores=NC, num_subcores=NS
    )


def _sc_degree(eib, ones_hbm, zeros_hbm):
    @functools.partial(
        pl.kernel,
        out_type=jax.ShapeDtypeStruct((NC, N_NODES, D_HID), jnp.float32),
        mesh=_vector_mesh(),
        compiler_params=pltpu.CompilerParams(use_tc_tiling_on_sc=False),
        scratch_types=[
            pltpu.VMEM((NCH, CHUNK), jnp.int32),
            pltpu.VMEM((CHUNK, D_HID), jnp.float32),
            pltpu.VMEM_SHARED((NPA, D_HID), jnp.float32),
            pltpu.SemaphoreType.DMA,
        ],
    )
    def deg_kernel(eib_hbm, ones_h, zeros_h, out_hbm, didx_v, ones_v, acc_s,
                   sem):
        c = lax.axis_index("c")
        s = lax.axis_index("s")
        wid = s * NC + c
        r0 = s * RPS
        st = [
            pltpu.async_copy(zeros_h.at[pl.ds(0, RPS)],
                             acc_s.at[pl.ds(r0, RPS)], sem),
            pltpu.async_copy(ones_h, ones_v, sem),
            pltpu.async_copy(eib_hbm.at[pl.ds(NCHT + wid * NCH, NCH)],
                             didx_v, sem),
        ]
        for cp in st:
            cp.wait()
        plsc.subcore_barrier()

        def fire8(g):
            for k in range(8):
                pltpu.async_copy(ones_v, acc_s.at[didx_v.at[g * 8 + k]], sem,
                                 add=True)

        def drain(n):
            for _ in range(n):
                pltpu.make_async_copy(out_hbm.at[0].at[pl.ds(0, CHUNK)],
                                      ones_v, sem).wait()

        fire8(0)
        fire8(1)

        @pl.loop(0, NCH // 8 - 2)
        def _(g):
            fire8(g + 2)
            drain(8)

        drain(16)
        plsc.subcore_barrier()
        pltpu.sync_copy(acc_s.at[pl.ds(r0, RPS)],
                        out_hbm.at[c].at[pl.ds(r0, RPS)])

    return deg_kernel(eib, ones_hbm, zeros_hbm)


def _sc_aggregate(hs_lin, eib, zeros_hbm):
    @functools.partial(
        pl.kernel,
        out_type=jax.ShapeDtypeStruct((NC, N_NODES, D_HID), jnp.float32),
        mesh=_vector_mesh(),
        compiler_params=pltpu.CompilerParams(use_tc_tiling_on_sc=False),
        scratch_types=[
            pltpu.VMEM((NCH, CHUNK), jnp.int32),
            pltpu.VMEM((NCH, CHUNK), jnp.int32),
            pltpu.VMEM((G, CHUNK, D_HID), jnp.float32),
            pltpu.VMEM((G, CHUNK, D_HID), jnp.float32),
            pltpu.VMEM_SHARED((NPA, D_HID), jnp.float32),
            pltpu.VMEM_SHARED((NPA, D_HID), jnp.float32),
            pltpu.SemaphoreType.DMA,
            pltpu.SemaphoreType.DMA,
            pltpu.SemaphoreType.DMA,
            pltpu.SemaphoreType.DMA,
        ],
    )
    def agg_kernel(hs_hbm, eib_hbm, zeros_h, out_hbm,
                   sidx_v, didx_v, rows_a, rows_b, hs_s, acc_s,
                   semga, semgb, semsa, semsb):
        c = lax.axis_index("c")
        s = lax.axis_index("s")
        wid = s * NC + c
        r0 = s * RPS
        st = [
            pltpu.async_copy(hs_hbm.at[pl.ds(r0, RPS)],
                             hs_s.at[pl.ds(r0, RPS)], semga),
            pltpu.async_copy(zeros_h.at[pl.ds(0, RPS)],
                             acc_s.at[pl.ds(r0, RPS)], semgb),
            pltpu.async_copy(eib_hbm.at[pl.ds(wid * NCH, NCH)], sidx_v,
                             semsa),
            pltpu.async_copy(eib_hbm.at[pl.ds(NCHT + wid * NCH, NCH)],
                             didx_v, semsb),
        ]
        for cp in st:
            cp.wait()
        plsc.subcore_barrier()

        def fire_g(g, rows, sem):
            for k in range(G):
                pltpu.async_copy(hs_s.at[sidx_v.at[g * G + k]], rows.at[k],
                                 sem)

        def fire_s(g, rows, sem):
            for k in range(G):
                pltpu.async_copy(rows.at[k], acc_s.at[didx_v.at[g * G + k]],
                                 sem, add=True)

        def drain(sem, n):
            for _ in range(n):
                pltpu.make_async_copy(hs_hbm.at[pl.ds(0, CHUNK)],
                                      rows_a.at[0], sem).wait()

        fire_g(0, rows_a, semga)
        fire_g(1, rows_b, semgb)
        drain(semga, G)
        fire_s(0, rows_a, semsa)

        @pl.loop(1, NGRP - 1, step=2)
        def _(g):
            drain(semsa, G)
            fire_g(g + 1, rows_a, semga)
            drain(semgb, G)
            fire_s(g, rows_b, semsb)
            drain(semsb, G)
            fire_g(g + 2, rows_b, semgb)
            drain(semga, G)
            fire_s(g + 1, rows_a, semsa)

        drain(semsa, G)
        drain(semgb, G)
        fire_s(NGRP - 1, rows_b, semsb)
        drain(semsb, G)

        plsc.subcore_barrier()
        pltpu.sync_copy(acc_s.at[pl.ds(r0, RPS)],
                        out_hbm.at[c].at[pl.ds(r0, RPS)])

    return agg_kernel(hs_lin, eib, zeros_hbm)


def _pk_spec():
    return pl.BlockSpec((PK, D_FEAT), lambda i: (0, 0))


def _pair_spec():
    return pl.BlockSpec((NC, PK, D_FEAT), lambda i: (0, 0, 0))


def _full_spec(shape):
    return pl.BlockSpec(shape, lambda i: tuple(0 for _ in shape))


def _mm1_body(x_ref, g_ref, w_ref, h_ref):
    x3 = x_ref[...]
    w = w_ref[...]
    parts = [
        jnp.dot(x3[:, j, :], w, preferred_element_type=jnp.float32)
        for j in range(8)
    ]
    h_ref[...] = jnp.concatenate(parts, axis=1) * g_ref[...]


def _scale_body(degp_ref, h_ref, dinv_ref, hs_ref):
    deg = degp_ref[0] + degp_ref[1] + 1.0
    dinv = lax.rsqrt(deg)
    dinv_ref[...] = dinv
    hs_ref[...] = h_ref[...] * dinv


def _relu_body(aggp_ref, hs1_ref, dinv_ref, b1_ref, hs2_ref):
    dinv = dinv_ref[...]
    conv1 = dinv * (aggp_ref[0] + aggp_ref[1] + hs1_ref[...]) + b1_ref[...]
    hs2_ref[...] = jnp.maximum(conv1, 0.0) * dinv


def _mm2_body(aggp_ref, hs2_ref, dinv_ref, w_ref, b_ref, o_ref):
    agg2 = dinv_ref[...] * (aggp_ref[0] + aggp_ref[1] + hs2_ref[...])
    w = w_ref[...]
    b = b_ref[...]
    for j in range(8):
        aj = agg2[:, 16 * j:16 * j + 16]
        o_ref[:, j, :] = (
            jnp.dot(aj, w, preferred_element_type=jnp.float32) + b
        )


def kernel(x, graph_seq, edge_index, W1, b1, W2, b2):
    nb, nc_, nd = x.shape
    n = nb * nc_

    xt = x.transpose(1, 0, 2)
    gs_pk = jnp.repeat(graph_seq.transpose(1, 0), D_HID, axis=1)

    pad_e = E_PAD - N_EDGES
    ei = edge_index.astype(jnp.int32)
    ei = (ei % nc_) * nb + ei // nc_
    ei = jnp.pad(ei, ((0, 0), (0, pad_e)))
    ei = ei.at[1, N_EDGES:].set(N_NODES)
    eib = ei.reshape(2 * NCHT, CHUNK)

    zeros = jnp.zeros((RPS, D_HID), jnp.float32)
    ones = jnp.ones((CHUNK, D_HID), jnp.float32)
    b1t = jnp.tile(b1, 8).reshape(1, D_FEAT)
    b2r = b2.reshape(1, D_FEAT)

    degp = _sc_degree(eib, ones, zeros).reshape(NC, PK, D_FEAT)

    h1 = pl.pallas_call(
        _mm1_body,
        grid=(1,),
        in_specs=[
            pl.BlockSpec((PK, nb, nd), lambda i: (0, 0, 0)),
            _full_spec((PK, D_FEAT)),
            _full_spec((D_FEAT, D_HID)),
        ],
        out_specs=_pk_spec(),
        out_shape=jax.ShapeDtypeStruct((PK, D_FEAT), jnp.float32),
    )(xt, gs_pk, W1)

    dinv, hs1 = pl.pallas_call(
        _scale_body,
        grid=(1,),
        in_specs=[_pair_spec(), _pk_spec()],
        out_specs=(_pk_spec(), _pk_spec()),
        out_shape=(
            jax.ShapeDtypeStruct((PK, D_FEAT), jnp.float32),
            jax.ShapeDtypeStruct((PK, D_FEAT), jnp.float32),
        ),
    )(degp, h1)

    agg1 = _sc_aggregate(hs1.reshape(n, D_HID), eib, zeros)
    agg1 = agg1.reshape(NC, PK, D_FEAT)

    hs2 = pl.pallas_call(
        _relu_body,
        grid=(1,),
        in_specs=[_pair_spec(), _pk_spec(), _pk_spec(),
                  _full_spec((1, D_FEAT))],
        out_specs=_pk_spec(),
        out_shape=jax.ShapeDtypeStruct((PK, D_FEAT), jnp.float32),
    )(agg1, hs1, dinv, b1t)

    agg2 = _sc_aggregate(hs2.reshape(n, D_HID), eib, zeros)
    agg2 = agg2.reshape(NC, PK, D_FEAT)

    out_t = pl.pallas_call(
        _mm2_body,
        grid=(1,),
        in_specs=[_pair_spec(), _pk_spec(), _pk_spec(),
                  _full_spec((D_HID, D_FEAT)), _full_spec((1, D_FEAT))],
        out_specs=pl.BlockSpec((PK, nb, nd), lambda i: (0, 0, 0)),
        out_shape=jax.ShapeDtypeStruct((PK, nb, nd), jnp.float32),
    )(agg2, hs2, dinv, W2, b2r)

    return out_t.transpose(1, 0, 2)
